# baseline (speedup 1.0000x reference)
import functools

import jax
import jax.numpy as jnp
from jax import lax
from jax.experimental import pallas as pl
from jax.experimental.pallas import tpu as pltpu
from jax.experimental.pallas import tpu_sc as plsc

_NODES = 10000
_D = 128
_E = 320000
_NC = 2
_NS = 16
_NW = _NC * _NS
_EW = _E // _NW
_C = 32
_NCH = 312
_TAIL = _EW - _NCH * _C


def _ipd_body(z_hbm, row_hbm, col_hbm, out_hbm,
              idxr_v, idxc_v, rr_v, rc_v, out_v, z_sp, gsems):
    cid = lax.axis_index("c")
    sid = lax.axis_index("s")
    wid = sid * _NC + cid
    wbase = wid * _EW

    iota = lax.iota(jnp.int32, 16)

    zrows = 624
    pltpu.sync_copy(z_hbm.at[pl.ds(sid * zrows, zrows)],
                    z_sp.at[pl.ds(sid * zrows, zrows)])

    @pl.when(sid == 0)
    def _():
        pltpu.sync_copy(z_hbm.at[pl.ds(_NS * zrows, _NODES - _NS * zrows)],
                        z_sp.at[pl.ds(_NS * zrows, _NODES - _NS * zrows)])

    pltpu.sync_copy(row_hbm.at[pl.ds(wbase, _EW)], idxr_v)
    pltpu.sync_copy(col_hbm.at[pl.ds(wbase, _EW)], idxc_v)

    plsc.subcore_barrier()

    def start(i, b, sz):
        idxr = idxr_v.at[pl.ds(i * _C, sz)]
        idxc = idxc_v.at[pl.ds(i * _C, sz)]
        dstr = rr_v.at[b, pl.ds(0, sz)]
        dstc = rc_v.at[b, pl.ds(0, sz)]
        pltpu.async_copy(z_sp.at[idxr], dstr, gsems.at[b])
        pltpu.async_copy(z_sp.at[idxc], dstc, gsems.at[b])

    def finish(i, b, sz):
        idxr = idxr_v.at[pl.ds(i * _C, sz)]
        idxc = idxc_v.at[pl.ds(i * _C, sz)]
        dstr = rr_v.at[b, pl.ds(0, sz)]
        dstc = rc_v.at[b, pl.ds(0, sz)]
        pltpu.make_async_copy(z_sp.at[idxr], dstr, gsems.at[b]).wait()
        pltpu.make_async_copy(z_sp.at[idxc], dstc, gsems.at[b]).wait()
        for g in range(sz // 16):

            def jbody(j, gacc):
                rrow = rr_v.at[b, g * 16 + j]
                crow = rc_v.at[b, g * 16 + j]
                acc = rrow[pl.ds(0, 16)] * crow[pl.ds(0, 16)]
                for k in range(1, _D // 16):
                    acc = acc + (rrow[pl.ds(k * 16, 16)]
                                 * crow[pl.ds(k * 16, 16)])
                s = jnp.sum(acc)
                return jnp.where(iota == j, s, gacc)

            group_acc = lax.fori_loop(0, 16, jbody,
                                      jnp.zeros((16,), jnp.float32),
                                      unroll=8)
            out_v[pl.ds(i * _C + g * 16, 16)] = (
                1.0 / (1.0 + jnp.exp(-group_acc)))

    start(0, 0, _C)

    def outer(j, carry):
        i0 = j * 2
        for b in range(2):
            i = i0 + b

            @pl.when(i + 1 < _NCH)
            def _():
                start(i + 1, 1 - b, _C)

            finish(i, b, _C)
        return carry

    lax.fori_loop(0, _NCH // 2, outer, 0)

    start(_NCH, 0, _TAIL)
    finish(_NCH, 0, _TAIL)

    pltpu.sync_copy(out_v, out_hbm.at[pl.ds(wbase, _EW)])


@jax.jit
def kernel(z, edge_index):
    ei = edge_index.astype(jnp.int32)
    row = ei[0]
    col = ei[1]
    mesh = plsc.VectorSubcoreMesh(
        core_axis_name="c", subcore_axis_name="s",
        num_cores=_NC, num_subcores=_NS)
    f = pl.kernel(
        _ipd_body,
        out_type=jax.ShapeDtypeStruct((_E,), jnp.float32),
        mesh=mesh,
        scratch_types=[
            pltpu.VMEM((_EW,), jnp.int32),
            pltpu.VMEM((_EW,), jnp.int32),
            pltpu.VMEM((2, _C, _D), jnp.float32),
            pltpu.VMEM((2, _C, _D), jnp.float32),
            pltpu.VMEM((_EW,), jnp.float32),
            pltpu.VMEM_SHARED((_NODES, _D), jnp.float32),
            pltpu.SemaphoreType.DMA((2,)),
        ],
        compiler_params=pltpu.CompilerParams(needs_layout_passes=False),
    )
    return f(z, row, col)


# speedup vs baseline: 1.1150x; 1.1150x over previous
import functools

import jax
import jax.numpy as jnp
from jax import lax
from jax.experimental import pallas as pl
from jax.experimental.pallas import tpu as pltpu
from jax.experimental.pallas import tpu_sc as plsc

_NODES = 10000
_D = 128
_E = 320000
_NC = 2
_NS = 16
_NW = _NC * _NS
_EW = _E // _NW
_C = 32
_NCH = 312
_TAIL = _EW - _NCH * _C


def _ipd_body(z_hbm, row_hbm, col_hbm, out_hbm,
              idxr_v, idxc_v, rr_v, rc_v, out_v, z_sp, gsems):
    cid = lax.axis_index("c")
    sid = lax.axis_index("s")
    wid = sid * _NC + cid
    wbase = wid * _EW

    iota = lax.iota(jnp.int32, 16)

    zrows = 624
    pltpu.sync_copy(z_hbm.at[pl.ds(sid * zrows, zrows)],
                    z_sp.at[pl.ds(sid * zrows, zrows)])

    @pl.when(sid == 0)
    def _():
        pltpu.sync_copy(z_hbm.at[pl.ds(_NS * zrows, _NODES - _NS * zrows)],
                        z_sp.at[pl.ds(_NS * zrows, _NODES - _NS * zrows)])

    pltpu.sync_copy(row_hbm.at[pl.ds(wbase, _EW)], idxr_v)
    pltpu.sync_copy(col_hbm.at[pl.ds(wbase, _EW)], idxc_v)

    plsc.subcore_barrier()

    def start(i, b, sz):
        idxr = idxr_v.at[pl.ds(i * _C, sz)]
        idxc = idxc_v.at[pl.ds(i * _C, sz)]
        dstr = rr_v.at[b, pl.ds(0, sz)]
        dstc = rc_v.at[b, pl.ds(0, sz)]
        pltpu.async_copy(z_sp.at[idxr], dstr, gsems.at[b])
        pltpu.async_copy(z_sp.at[idxc], dstc, gsems.at[b])

    def finish(i, b, sz):
        idxr = idxr_v.at[pl.ds(i * _C, sz)]
        idxc = idxc_v.at[pl.ds(i * _C, sz)]
        dstr = rr_v.at[b, pl.ds(0, sz)]
        dstc = rc_v.at[b, pl.ds(0, sz)]
        pltpu.make_async_copy(z_sp.at[idxr], dstr, gsems.at[b]).wait()
        pltpu.make_async_copy(z_sp.at[idxc], dstc, gsems.at[b]).wait()
        for g in range(sz // 16):

            def jbody(j, gacc):
                rrow = rr_v.at[b, g * 16 + j]
                crow = rc_v.at[b, g * 16 + j]
                acc = rrow[pl.ds(0, 16)] * crow[pl.ds(0, 16)]
                for k in range(1, _D // 16):
                    acc = acc + (rrow[pl.ds(k * 16, 16)]
                                 * crow[pl.ds(k * 16, 16)])
                s = jnp.sum(acc)
                return jnp.where(iota == j, s, gacc)

            group_acc = lax.fori_loop(0, 16, jbody,
                                      jnp.zeros((16,), jnp.float32),
                                      unroll=2)
            out_v[pl.ds(i * _C + g * 16, 16)] = (
                1.0 / (1.0 + jnp.exp(-group_acc)))

    start(0, 0, _C)

    def outer(j, carry):
        i0 = j * 2
        for b in range(2):
            i = i0 + b

            @pl.when(i + 1 < _NCH)
            def _():
                start(i + 1, 1 - b, _C)

            finish(i, b, _C)
        return carry

    lax.fori_loop(0, _NCH // 2, outer, 0)

    start(_NCH, 0, _TAIL)
    finish(_NCH, 0, _TAIL)

    pltpu.sync_copy(out_v, out_hbm.at[pl.ds(wbase, _EW)])


@jax.jit
def kernel(z, edge_index):
    ei = edge_index.astype(jnp.int32)
    row = ei[0]
    col = ei[1]
    mesh = plsc.VectorSubcoreMesh(
        core_axis_name="c", subcore_axis_name="s",
        num_cores=_NC, num_subcores=_NS)
    f = pl.kernel(
        _ipd_body,
        out_type=jax.ShapeDtypeStruct((_E,), jnp.float32),
        mesh=mesh,
        scratch_types=[
            pltpu.VMEM((_EW,), jnp.int32),
            pltpu.VMEM((_EW,), jnp.int32),
            pltpu.VMEM((2, _C, _D), jnp.float32),
            pltpu.VMEM((2, _C, _D), jnp.float32),
            pltpu.VMEM((_EW,), jnp.float32),
            pltpu.VMEM_SHARED((_NODES, _D), jnp.float32),
            pltpu.SemaphoreType.DMA((2,)),
        ],
        compiler_params=pltpu.CompilerParams(needs_layout_passes=False),
    )
    return f(z, row, col)
